# baseline (speedup 1.0000x reference)
import functools

import jax
import jax.numpy as jnp
from jax.experimental import pallas as pl
from jax.experimental.pallas import tpu as pltpu

_ETA_OPT = 1000000.0


def _body(a_ref, at_ref, q_ref, x_ref, y_ref, b_ref, c_ref, iy_ref,
          il_ref, iu_ref, l_ref, u_ref, out_ref, acc_ref, *, blk, nsteps):
    i = pl.program_id(0)

    xv = x_ref[...]
    yv = y_ref[...]

    ax = jnp.dot(a_ref[...], xv, preferred_element_type=jnp.float32)
    y_blk = y_ref[pl.ds(i * blk, blk), :]
    axb = y_blk * (ax - b_ref[...]) * iy_ref[...]
    s = jnp.sum(jnp.abs(axb))

    aty = jnp.dot(at_ref[...], yv, preferred_element_type=jnp.float32)
    qx = jnp.dot(q_ref[...], xv, preferred_element_type=jnp.float32)
    pg = c_ref[...] - aty + qx

    x_blk = x_ref[pl.ds(i * blk, blk), :]
    lb = jnp.maximum(pg, 0.0) * il_ref[...]
    s = s + jnp.sum(jnp.abs((x_blk - l_ref[...]) * lb))
    ub = jnp.maximum(-pg, 0.0) * iu_ref[...]
    s = s + jnp.sum(jnp.abs((u_ref[...] - x_blk) * ub))

    @pl.when(i == 0)
    def _():
        acc_ref[0] = 0.0

    acc_ref[0] += s

    @pl.when(i == nsteps - 1)
    def _():
        out_ref[...] = jnp.full((1, 1), acc_ref[0] * (1.0 / _ETA_OPT),
                                dtype=jnp.float32)


def kernel(Q, A, AT, b, c, x, y, Iy, il, iu, l, u):
    n = Q.shape[0]
    m = A.shape[0]
    blk = 512
    nsteps = n // blk

    b2 = b[:, None]
    c2 = c[:, None]

    row_spec = lambda cols: pl.BlockSpec((blk, cols), lambda i: (i, 0))
    vec_spec = pl.BlockSpec((blk, 1), lambda i: (i, 0))
    full_spec = lambda rows: pl.BlockSpec((rows, 1), lambda i: (0, 0))

    out = pl.pallas_call(
        functools.partial(_body, blk=blk, nsteps=nsteps),
        grid=(nsteps,),
        in_specs=[
            row_spec(n),
            row_spec(m),
            row_spec(n),
            full_spec(n),
            full_spec(m),
            vec_spec,
            vec_spec,
            vec_spec,
            vec_spec,
            vec_spec,
            vec_spec,
            vec_spec,
        ],
        out_specs=pl.BlockSpec((1, 1), lambda i: (0, 0)),
        out_shape=jax.ShapeDtypeStruct((1, 1), jnp.float32),
        scratch_shapes=[pltpu.SMEM((1,), jnp.float32)],
    )(A, AT, Q, x, y, b2, c2, Iy, il, iu, l, u)
    return out[0, 0]


# speedup vs baseline: 1.0277x; 1.0277x over previous
import functools

import jax
import jax.numpy as jnp
from jax.experimental import pallas as pl
from jax.experimental.pallas import tpu as pltpu

_ETA_OPT = 1000000.0


def _body(a_ref, at_ref, q_ref, x_ref, y_ref, b_ref, c_ref, iy_ref,
          il_ref, iu_ref, l_ref, u_ref, out_ref, acc_ref, *, blk, nsteps):
    i = pl.program_id(0)

    xv = x_ref[...]
    yv = y_ref[...]

    ax = jnp.dot(a_ref[...], xv, preferred_element_type=jnp.float32)
    y_blk = y_ref[pl.ds(i * blk, blk), :]
    axb = y_blk * (ax - b_ref[...]) * iy_ref[...]
    s = jnp.sum(jnp.abs(axb))

    aty = jnp.dot(at_ref[...], yv, preferred_element_type=jnp.float32)
    qx = jnp.dot(q_ref[...], xv, preferred_element_type=jnp.float32)
    pg = c_ref[...] - aty + qx

    x_blk = x_ref[pl.ds(i * blk, blk), :]
    lb = jnp.maximum(pg, 0.0) * il_ref[...]
    s = s + jnp.sum(jnp.abs((x_blk - l_ref[...]) * lb))
    ub = jnp.maximum(-pg, 0.0) * iu_ref[...]
    s = s + jnp.sum(jnp.abs((u_ref[...] - x_blk) * ub))

    @pl.when(i == 0)
    def _():
        acc_ref[0] = 0.0

    acc_ref[0] += s

    @pl.when(i == nsteps - 1)
    def _():
        out_ref[...] = jnp.full((1, 1), acc_ref[0] * (1.0 / _ETA_OPT),
                                dtype=jnp.float32)


def kernel(Q, A, AT, b, c, x, y, Iy, il, iu, l, u):
    n = Q.shape[0]
    m = A.shape[0]
    blk = 256
    nsteps = n // blk

    b2 = b[:, None]
    c2 = c[:, None]

    row_spec = lambda cols: pl.BlockSpec((blk, cols), lambda i: (i, 0))
    vec_spec = pl.BlockSpec((blk, 1), lambda i: (i, 0))
    full_spec = lambda rows: pl.BlockSpec((rows, 1), lambda i: (0, 0))

    out = pl.pallas_call(
        functools.partial(_body, blk=blk, nsteps=nsteps),
        grid=(nsteps,),
        in_specs=[
            row_spec(n),
            row_spec(m),
            row_spec(n),
            full_spec(n),
            full_spec(m),
            vec_spec,
            vec_spec,
            vec_spec,
            vec_spec,
            vec_spec,
            vec_spec,
            vec_spec,
        ],
        out_specs=pl.BlockSpec((1, 1), lambda i: (0, 0)),
        out_shape=jax.ShapeDtypeStruct((1, 1), jnp.float32),
        scratch_shapes=[pltpu.SMEM((1,), jnp.float32)],
    )(A, AT, Q, x, y, b2, c2, Iy, il, iu, l, u)
    return out[0, 0]
